# baseline (speedup 1.0000x reference)
import jax
import jax.numpy as jnp
from jax.experimental import pallas as pl
from jax.experimental.pallas import tpu as pltpu

N = 32768
DIM = 64
K = 1024
BN = 8192
BNH = BN // 2


def _assign_kernel(xr_ref, cr_ref, ae_ref, ao_ref):
    xr = xr_ref[...]
    cr = cr_ref[...]
    c = jnp.stack([cr[:, :DIM], cr[:, DIM:]], axis=1).reshape(K, DIM)
    c2 = jnp.sum(c * c, axis=1)
    cneg = c * (-2.0)
    xe = xr[:, :DIM]
    xo = xr[:, DIM:]
    se = jax.lax.dot_general(
        cneg, xe, (((1,), (1,)), ((), ())),
        preferred_element_type=jnp.float32) + c2[:, None]
    so = jax.lax.dot_general(
        cneg, xo, (((1,), (1,)), ((), ())),
        preferred_element_type=jnp.float32) + c2[:, None]
    ae_ref[...] = jnp.argmin(se, axis=0).astype(jnp.int32)
    ao_ref[...] = jnp.argmin(so, axis=0).astype(jnp.int32)


@jax.jit
def kernel(x, cluster_centers):
    xr = x.reshape(N // 2, 2 * DIM)
    cr = cluster_centers.reshape(K // 2, 2 * DIM)
    ae, ao = pl.pallas_call(
        _assign_kernel,
        grid=(N // BN,),
        in_specs=[
            pl.BlockSpec((BNH, 2 * DIM), lambda i: (i, 0)),
            pl.BlockSpec((K // 2, 2 * DIM), lambda i: (0, 0)),
        ],
        out_specs=[
            pl.BlockSpec((BNH,), lambda i: (i,)),
            pl.BlockSpec((BNH,), lambda i: (i,)),
        ],
        out_shape=[
            jax.ShapeDtypeStruct((N // 2,), jnp.int32),
            jax.ShapeDtypeStruct((N // 2,), jnp.int32),
        ],
        compiler_params=pltpu.CompilerParams(
            dimension_semantics=("parallel",)),
    )(xr, cr)
    return jnp.stack([ae, ao], axis=1).reshape(N)


# speedup vs baseline: 3.4318x; 3.4318x over previous
import jax
import jax.numpy as jnp
from jax.experimental import pallas as pl
from jax.experimental.pallas import tpu as pltpu

N = 32768
DIM = 64
K = 1024
BN = 8192


def _assign_kernel(xt_ref, ct_ref, out_ref):
    xt = xt_ref[...]
    ct = ct_ref[...]
    c2 = jnp.sum(ct * ct, axis=0)
    ctneg = ct * (-2.0)
    dot = jax.lax.dot_general(
        ctneg, xt, (((0,), (0,)), ((), ())),
        preferred_element_type=jnp.float32)
    s = dot + c2[:, None]
    out_ref[...] = jnp.argmin(s, axis=0).astype(jnp.int32)


@jax.jit
def kernel(x, cluster_centers):
    return pl.pallas_call(
        _assign_kernel,
        grid=(N // BN,),
        in_specs=[
            pl.BlockSpec((DIM, BN), lambda i: (0, i)),
            pl.BlockSpec((DIM, K), lambda i: (0, 0)),
        ],
        out_specs=pl.BlockSpec((BN,), lambda i: (i,)),
        out_shape=jax.ShapeDtypeStruct((N,), jnp.int32),
        compiler_params=pltpu.CompilerParams(
            dimension_semantics=("parallel",)),
    )(x.T, cluster_centers.T)
